# Initial kernel scaffold; baseline (speedup 1.0000x reference)
import jax
import jax.numpy as jnp
from jax.experimental import pallas as pl


def kernel(patches, output):
    raise NotImplementedError("write your pallas kernel here")


# speedup vs baseline: 3.3910x; 3.3910x over previous
import functools

import jax
import jax.numpy as jnp
from jax import lax
from jax.experimental import pallas as pl
from jax.experimental.pallas import tpu as pltpu
from jax.experimental.pallas import tpu_sc as plsc

H = 4096
W = 4096
P = 512
NW = 32


def _stitch_body(pat_hbm, out_hbm):
    cid = lax.axis_index("c")
    sid = lax.axis_index("s")
    wid = sid * 2 + cid

    def run_variant(n_items, ch, cw, decode):
        trips = (n_items + NW - 1) // NW

        def scoped(buf):
            def trip(t, carry):
                j = wid + NW * t

                @pl.when(j < n_items)
                def _():
                    rc = j & 3
                    cell = j >> 2
                    sr, sc, dr, dc = decode(cell, rc)
                    pltpu.sync_copy(
                        pat_hbm.at[pl.ds(sr, ch), pl.ds(sc, cw)], buf
                    )
                    pltpu.sync_copy(
                        buf, out_hbm.at[pl.ds(dr, ch), pl.ds(dc, cw)]
                    )

                return carry

            lax.fori_loop(0, trips, trip, 0, unroll=False)

        pl.run_scoped(scoped, pltpu.VMEM((ch, cw), jnp.float32))

    def dec1(cell, rc):
        iy = cell >> 1
        ix = cell & 1
        sr = iy * 36896 + ix * 4096 + rc * 120
        sc = ix * 32
        dr = iy * 3616 + rc * 120
        dc = ix * 3616
        return sr, sc, dr, dc

    def dec2(cell, rc):
        iy = cell & 1
        ix = (cell >> 1) + 1
        sr = iy * 36896 + ix * 512 + rc * 120
        dr = iy * 3616 + rc * 120
        dc = ix * 448 + 32
        return sr, 32, dr, dc

    def dec3(cell, rc):
        ix = cell & 1
        iy = (cell >> 1) + 1
        sr = (iy * 9 + ix * 8) * 512 + 32 + rc * 112
        sc = ix * 32
        dr = iy * 448 + 32 + rc * 112
        dc = ix * 3616
        return sr, sc, dr, dc

    def dec4(cell, rc):
        iy = (cell * 37) >> 8
        ix = cell - iy * 7
        sr = ((iy + 1) * 9 + ix + 1) * 512 + 32 + rc * 112
        dr = (iy + 1) * 448 + 32 + rc * 112
        dc = (ix + 1) * 448 + 32
        return sr, 32, dr, dc

    run_variant(16, 120, 480, dec1)
    run_variant(56, 120, 448, dec2)
    run_variant(56, 112, 480, dec3)
    run_variant(196, 112, 448, dec4)


_stitch = functools.partial(
    pl.kernel,
    out_type=jax.ShapeDtypeStruct((H, W), jnp.float32),
    mesh=plsc.VectorSubcoreMesh(core_axis_name="c", subcore_axis_name="s"),
    compiler_params=pltpu.CompilerParams(use_tc_tiling_on_sc=False),
)(_stitch_body)


@jax.jit
def kernel(patches, output):
    del output
    pat = patches.reshape(81 * P, P)
    out = _stitch(pat)
    return out.reshape(1, 1, H, W)


# speedup vs baseline: 7.4276x; 2.1904x over previous
import functools

import jax
import jax.numpy as jnp
from jax import lax
from jax.experimental import pallas as pl
from jax.experimental.pallas import tpu as pltpu
from jax.experimental.pallas import tpu_sc as plsc

H = 4096
W = 4096
P = 512
NW = 32


def _stitch_body(pt_hbm, ot_hbm):
    cid = lax.axis_index("c")
    sid = lax.axis_index("s")
    wid = sid * 2 + cid

    def run_variant(nslots, ntr, ntc, nl, decode):
        trips = (nslots + NW - 1) // NW

        def scoped(buf):
            def trip(t, carry):
                j = wid + NW * t
                valid, p, ab, gt0, stc, sl0, dtc, dl0 = decode(j)
                a0 = p * 64 + ab

                @pl.when(jnp.logical_and(j < nslots, valid))
                def _():
                    pltpu.sync_copy(
                        pt_hbm.at[
                            pl.ds(a0, ntr), pl.ds(stc, ntc), :, pl.ds(sl0, nl)
                        ],
                        buf,
                    )
                    pltpu.sync_copy(
                        buf,
                        ot_hbm.at[
                            pl.ds(gt0, ntr), pl.ds(dtc, ntc), :, pl.ds(dl0, nl)
                        ],
                    )

                return carry

            lax.fori_loop(0, trips, trip, 0, unroll=False)

        pl.run_scoped(scoped, pltpu.VMEM((ntr, ntc, 8, nl), jnp.float32))

    def interior_rows(j):
        band = j >> 5
        chunk = j & 3
        iy = band + 1
        gt0 = 56 * iy + 4 + 14 * chunk
        ab = 4 + 14 * chunk
        return band < 7, iy, gt0, ab

    def edge_rows(j):
        bb = (j >> 5) & 1
        chunk = j & 3
        iy = 8 * bb
        gt0 = 452 * bb + 15 * chunk
        ab = 4 * bb + 15 * chunk
        return j >= 0, iy, gt0, ab

    def dec_a(rows):
        def d(j):
            rv, iy, gt0, ab = rows(j)
            which = (j >> 4) & 1
            ixo = (j >> 2) & 3
            ix = 2 * ixo + 1
            k = (7 * ix - 1) >> 1
            p = iy * 9 + ix
            return rv, p, ab, gt0, which, 64 - 64 * which, k + 1, 64 * which

        return d

    def dec_b(rows):
        def d(j):
            rv, iy, gt0, ab = rows(j)
            which = (j >> 4) & 1
            ixo = (j >> 2) & 3
            ix = 2 * ixo + 1
            k = (7 * ix - 1) >> 1
            p = iy * 9 + ix
            return (
                rv,
                p,
                ab,
                gt0,
                3 - 3 * which,
                64 - 32 * which,
                k + 4 - 4 * which,
                96 * which,
            )

        return d

    def dec_c(rows):
        def d(j):
            rv, iy, gt0, ab = rows(j)
            which = (j >> 4) & 1
            ixo = (j >> 2) & 3
            ix = 2 * ixo + 2 - 2 * which
            p = iy * 9 + ix
            return (
                rv,
                p,
                ab,
                gt0,
                3 * which,
                32 - 32 * which,
                7 * ixo + 7 - 4 * which,
                32 - 32 * which,
            )

        return d

    def dec_d(rows):
        def d(j):
            rv, iy, gt0, ab = rows(j)
            ixo = (j >> 2) & 3
            ix = 2 * ixo + 2
            p = iy * 9 + ix
            return jnp.logical_and(rv, ixo < 3), p, ab, gt0, 1, 0, 7 * ixo + 8, 0

        return d

    def dec_e(rows):
        def d(j):
            rv, iy, gt0, ab = rows(j)
            which = (j >> 2) & 1
            p = iy * 9 + 8 * which
            return rv, p, ab, gt0, which, 0, 29 * which, 0

        return d

    run_variant(256, 14, 3, 64, dec_a(interior_rows))
    run_variant(256, 14, 1, 32, dec_b(interior_rows))
    run_variant(256, 14, 1, 96, dec_c(interior_rows))
    run_variant(128, 14, 2, 128, dec_d(lambda j: interior_rows(((j >> 4) << 5) | (j & 15))))
    run_variant(64, 14, 3, 128, dec_e(lambda j: interior_rows(((j >> 3) << 5) | (j & 7))))
    run_variant(64, 15, 3, 64, dec_a(edge_rows))
    run_variant(64, 15, 1, 32, dec_b(edge_rows))
    run_variant(64, 15, 1, 96, dec_c(edge_rows))
    run_variant(32, 15, 2, 128, dec_d(lambda j: edge_rows(((j >> 4) << 5) | (j & 15))))
    run_variant(16, 15, 3, 128, dec_e(lambda j: edge_rows(((j >> 3) << 5) | (j & 7))))


_stitch = functools.partial(
    pl.kernel,
    out_type=jax.ShapeDtypeStruct((512, 32, 8, 128), jnp.float32),
    mesh=plsc.VectorSubcoreMesh(core_axis_name="c", subcore_axis_name="s"),
    compiler_params=pltpu.CompilerParams(use_tc_tiling_on_sc=False),
)(_stitch_body)


@jax.jit
def kernel(patches, output):
    del output
    pt = (
        patches.reshape(81, 64, 8, 4, 128)
        .transpose(0, 1, 3, 2, 4)
        .reshape(5184, 4, 8, 128)
    )
    ot = _stitch(pt)
    img = ot.transpose(0, 2, 1, 3).reshape(4096, 4096)
    return img.reshape(1, 1, H, W)
